# baseline (speedup 1.0000x reference)
import functools

import jax
import jax.numpy as jnp
from jax import lax
from jax.experimental import pallas as pl
from jax.experimental.pallas import tpu as pltpu
from jax.experimental.pallas import tpu_sc as plsc

GAMMA = -0.1
ZETA = 1.1
LANES = 16
EB = 128


def _node_scores_body(x_ref, wl_ref, wr_ref, bl_ref, br_ref, wa_ref, ba_ref,
                      st_ref):
    x = x_ref[...]
    h = wl_ref.shape[0]
    dn_tt = (((1,), (1,)), ((), ()))
    dn_nn = (((1,), (0,)), ((), ()))
    bl = lax.broadcast_in_dim(bl_ref[...], (h, 1), (0,))
    br = lax.broadcast_in_dim(br_ref[...], (h, 1), (0,))
    gl = jnp.maximum(
        lax.dot_general(wl_ref[...], x, dn_tt,
                        preferred_element_type=jnp.float32) + bl, 0.0)
    gr = jnp.maximum(
        lax.dot_general(wr_ref[...], x, dn_tt,
                        preferred_element_type=jnp.float32) + br, 0.0)
    sl_row = lax.dot_general(wa_ref[:, :h], gl, dn_nn,
                             preferred_element_type=jnp.float32) + ba_ref[0]
    sr_row = lax.dot_general(wa_ref[:, h:], gr, dn_nn,
                             preferred_element_type=jnp.float32)
    st_ref[...] = jnp.concatenate([sl_row, sr_row], axis=0)


def _node_scores(x, W_l, b_l, W_r, b_r, W_a, b_a):
    n, d = x.shape
    h = W_l.shape[0]
    st = pl.pallas_call(
        _node_scores_body,
        out_shape=jax.ShapeDtypeStruct((2, n), jnp.float32),
    )(x, W_l, W_r, b_l, b_r, W_a, b_a)
    return st


def _edge_gate(st, edge_index, noise, adj_values):
    n = st.shape[1]
    e = noise.shape[0]
    info = plsc.get_sparse_core_info()
    nc, ns = info.num_cores, info.num_subcores
    nw = nc * ns
    nblk = e // EB
    assert nblk * EB == e
    per = nblk // nw
    main = per * EB
    rem = nblk - per * nw
    cap = main + (EB if rem else 0)
    assert rem <= nw

    nch = 3
    csz = main // nch
    assert csz * nch == main and csz % EB == 0 and (csz // LANES) % 16 == 0

    mesh = plsc.VectorSubcoreMesh(core_axis_name="c", subcore_axis_name="s")

    @functools.partial(
        pl.kernel,
        out_type=jax.ShapeDtypeStruct((e,), jnp.float32),
        mesh=mesh,
        compiler_params=pltpu.CompilerParams(needs_layout_passes=False),
        scratch_types=[
            pltpu.VMEM((2, n), jnp.float32),
            pltpu.VMEM((2, cap), jnp.int32),
            pltpu.VMEM((cap,), jnp.float32),
            pltpu.VMEM((cap,), jnp.float32),
            pltpu.VMEM((cap,), jnp.float32),
            pltpu.SemaphoreType.DMA,
            pltpu.SemaphoreType.DMA,
            [pltpu.SemaphoreType.DMA] * nch,
            [pltpu.SemaphoreType.DMA] * nch,
            [pltpu.SemaphoreType.DMA] * nch,
        ],
    )
    def run(st_hbm, ei_hbm, noise_hbm, adj_hbm, out_hbm,
            st_v, ei_v, noise_v, adj_v, out_v,
            sem_st, sem_out, sems_ei, sems_no, sems_ad):
        wid = lax.axis_index("s") * nc + lax.axis_index("c")
        c0 = pl.multiple_of(wid * main, EB)
        x0 = pl.multiple_of(nw * main + wid * EB, EB)

        cp_st = pltpu.async_copy(st_hbm, st_v, sem_st)
        cps = []
        for k in range(nch):
            o = k * csz
            cps.append((
                pltpu.async_copy(ei_hbm.at[:, pl.ds(c0 + o, csz)],
                                 ei_v.at[:, pl.ds(o, csz)], sems_ei[k]),
                pltpu.async_copy(noise_hbm.at[pl.ds(c0 + o, csz)],
                                 noise_v.at[pl.ds(o, csz)], sems_no[k]),
                pltpu.async_copy(adj_hbm.at[pl.ds(c0 + o, csz)],
                                 adj_v.at[pl.ds(o, csz)], sems_ad[k]),
            ))

        @pl.when(wid < rem)
        def _():
            pltpu.async_copy(ei_hbm.at[:, pl.ds(x0, EB)],
                             ei_v.at[:, pl.ds(main, EB)], sems_ei[0]).wait()
            pltpu.async_copy(noise_hbm.at[pl.ds(x0, EB)],
                             noise_v.at[pl.ds(main, EB)], sems_no[0]).wait()
            pltpu.async_copy(adj_hbm.at[pl.ds(x0, EB)],
                             adj_v.at[pl.ds(main, EB)], sems_ad[0]).wait()

        zero16 = jnp.zeros((LANES,), jnp.int32)
        one16 = jnp.ones((LANES,), jnp.int32)

        def gate_at(off):
            r = ei_v[0, pl.ds(off, LANES)]
            c = ei_v[1, pl.ds(off, LANES)]
            a = plsc.load_gather(st_v, [zero16, r])
            b = plsc.load_gather(st_v, [one16, c])
            u = noise_v[pl.ds(off, LANES)]
            t = jnp.exp(-(a + b))
            gate = u / (u + (1.0 - u) * t)
            m = jnp.minimum(jnp.maximum(gate * (ZETA - GAMMA) + GAMMA, 0.0), 1.0)
            out_v[pl.ds(off, LANES)] = adj_v[pl.ds(off, LANES)] * m

        cp_st.wait()
        out_cps = []
        for k in range(nch):
            for cp in cps[k]:
                cp.wait()
            o = k * csz
            plsc.parallel_loop(o, o + csz, LANES, unroll=16)(gate_at)
            out_cps.append(
                pltpu.async_copy(out_v.at[pl.ds(o, csz)],
                                 out_hbm.at[pl.ds(c0 + o, csz)], sem_out))

        @pl.when(wid < rem)
        def _():
            plsc.parallel_loop(main, main + EB, LANES, unroll=8)(gate_at)
            pltpu.async_copy(out_v.at[pl.ds(main, EB)],
                             out_hbm.at[pl.ds(x0, EB)], sem_out).wait()

        for cp in out_cps:
            cp.wait()

    return run(st, edge_index, noise, adj_values)


def kernel(x, edge_index, adj_values, noise, W_l, b_l, W_r, b_r, W_a, b_a):
    st = _node_scores(x, W_l, b_l, W_r, b_r, W_a, b_a)
    return _edge_gate(st, edge_index, noise, adj_values)
